# baseline (speedup 1.0000x reference)
import functools

import jax
import jax.numpy as jnp
from jax import lax
from jax.experimental import pallas as pl
from jax.experimental.pallas import tpu as pltpu
from jax.experimental.pallas import tpu_sc as plsc

N = 10000
E = 320000
D_IN = 128
D_H = 128
D_OUT = 64

NC = 2
NS = 16
NW = NC * NS
EPW = E // NW
C = 128
EPW_P = 10240
CPW = EPW_P // C
W = 20
NWIN = CPW // W
N_PAD = 10240
ROWS_PT = N_PAD // NS

@functools.cache
def _make_agg():
    mesh = plsc.VectorSubcoreMesh(core_axis_name="c", subcore_axis_name="s")

    @functools.partial(
        pl.kernel,
        out_type=jax.ShapeDtypeStruct((NC * N_PAD, D_H), jnp.float32),
        mesh=mesh,
        scratch_types=[
            pltpu.VMEM((2, W, C), jnp.int32),
            pltpu.VMEM((2, W, 2, C // 2), jnp.int32),
            pltpu.VMEM((2, C, D_H), jnp.float32),
            pltpu.VMEM_SHARED((N_PAD, D_H), jnp.float32),
            pltpu.SemaphoreType.DMA,
            pltpu.SemaphoreType.DMA,
            pltpu.SemaphoreType.DMA,
            pltpu.SemaphoreType.DMA,
            pltpu.SemaphoreType.DMA,
            pltpu.SemaphoreType.DMA,
        ],
    )
    def _agg(src_hbm, dst_hbm, h_hbm, zeros_hbm, out_hbm,
             srcv, dstv, rows, acc, sem0, sem1, semi, semz, semsa, semsb):

        def scat2(buf, dw, j):
            half = C // 2
            ra = rows.at[buf].at[pl.ds(0, half)]
            rb = rows.at[buf].at[pl.ds(half, half)]
            pltpu.async_copy(ra, acc.at[dw.at[j, 0]], semsa, add=True)
            pltpu.async_copy(rb, acc.at[dw.at[j, 1]], semsb, add=True)
            pltpu.make_async_copy(ra, acc.at[dw.at[j, 0]], semsa).wait()
            pltpu.make_async_copy(rb, acc.at[dw.at[j, 1]], semsb).wait()
        cid = lax.axis_index("c")
        sid = lax.axis_index("s")
        wid = cid * NS + sid

        pltpu.async_copy(zeros_hbm, acc.at[pl.ds(sid * ROWS_PT, ROWS_PT)],
                         semz)
        pltpu.async_copy(src_hbm.at[wid, 0], srcv.at[0], semi)
        pltpu.async_copy(dst_hbm.at[wid, 0], dstv.at[0], semi)
        pltpu.make_async_copy(src_hbm.at[wid, 0], srcv.at[0], semi).wait()
        pltpu.make_async_copy(dst_hbm.at[wid, 0], dstv.at[0], semi).wait()
        pltpu.async_copy(h_hbm.at[srcv.at[0].at[0]], rows.at[0], sem0)
        pltpu.async_copy(h_hbm.at[srcv.at[0].at[1]], rows.at[1], sem1)
        pltpu.make_async_copy(zeros_hbm,
                              acc.at[pl.ds(sid * ROWS_PT, ROWS_PT)],
                              semz).wait()
        plsc.subcore_barrier()

        for w in range(NWIN):
            pb = w % 2
            sw = srcv.at[pb]
            dw = dstv.at[pb]
            if w >= 1:
                pltpu.make_async_copy(src_hbm.at[wid, w], sw, semi).wait()
                pltpu.make_async_copy(dst_hbm.at[wid, w], dw, semi).wait()
            if w + 1 < NWIN:
                nb = (w + 1) % 2
                pltpu.async_copy(src_hbm.at[wid, w + 1], srcv.at[nb], semi)
                pltpu.async_copy(dst_hbm.at[wid, w + 1], dstv.at[nb], semi)

            if w >= 1:
                pltpu.async_copy(h_hbm.at[sw.at[0]], rows.at[0], sem0)
                pltpu.async_copy(h_hbm.at[sw.at[1]], rows.at[1], sem1)

            def body(i, carry, sw=sw, dw=dw):
                j0 = 2 * i
                j1 = j0 + 1
                pltpu.make_async_copy(h_hbm.at[sw.at[j0]], rows.at[0],
                                      sem0).wait()
                scat2(0, dw, j0)
                pltpu.async_copy(h_hbm.at[sw.at[j0 + 2]], rows.at[0], sem0)
                pltpu.make_async_copy(h_hbm.at[sw.at[j1]], rows.at[1],
                                      sem1).wait()
                scat2(1, dw, j1)
                pltpu.async_copy(h_hbm.at[sw.at[j1 + 2]], rows.at[1], sem1)
                return carry

            n_steady = (W - 2) // 2
            lax.fori_loop(0, n_steady, body, 0)
            j_a = 2 * n_steady
            pltpu.make_async_copy(h_hbm.at[sw.at[j_a]], rows.at[0],
                                  sem0).wait()
            scat2(0, dw, j_a)
            if W % 2:
                pltpu.async_copy(h_hbm.at[sw.at[W - 1]], rows.at[0], sem0)
            pltpu.make_async_copy(h_hbm.at[sw.at[j_a + 1]], rows.at[1],
                                  sem1).wait()
            scat2(1, dw, j_a + 1)
            if W % 2:
                pltpu.make_async_copy(h_hbm.at[sw.at[W - 1]], rows.at[0],
                                      sem0).wait()
                scat2(0, dw, W - 1)
        plsc.subcore_barrier()
        pltpu.sync_copy(acc.at[pl.ds(sid * ROWS_PT, ROWS_PT)],
                        out_hbm.at[pl.ds(cid * N_PAD + sid * ROWS_PT, ROWS_PT)])

    return _agg


def _dense1_body(x_ref, p_ref, wa_ref, ba_ref, g_ref, be_ref, wb_ref, bb_ref,
                 out_ref):
    z = x_ref[...] + p_ref[0:N, :] + p_ref[N_PAD:N_PAD + N, :]
    t = jnp.dot(z, wa_ref[...], preferred_element_type=jnp.float32) + ba_ref[...]
    mu = jnp.mean(t, axis=0, keepdims=True)
    var = jnp.mean((t - mu) ** 2, axis=0, keepdims=True)
    y = g_ref[...] * (t - mu) * lax.rsqrt(var + 1e-5) + be_ref[...]
    y = jnp.maximum(y, 0.0)
    h = jnp.dot(y, wb_ref[...], preferred_element_type=jnp.float32) + bb_ref[...]
    out_ref[...] = jnp.maximum(h, 0.0)


_dense1 = pl.pallas_call(
    _dense1_body,
    out_shape=jax.ShapeDtypeStruct((N, D_H), jnp.float32),
)


def _dense2_body(x_ref, p_ref, wa_ref, ba_ref, g_ref, be_ref, wb_ref, bb_ref,
                 w3a_ref, b3a_ref, w3b_ref, b3b_ref, out_ref):
    z = x_ref[...] + p_ref[0:N, :] + p_ref[N_PAD:N_PAD + N, :]
    t = jnp.dot(z, wa_ref[...], preferred_element_type=jnp.float32) + ba_ref[...]
    mu = jnp.mean(t, axis=0, keepdims=True)
    var = jnp.mean((t - mu) ** 2, axis=0, keepdims=True)
    y = g_ref[...] * (t - mu) * lax.rsqrt(var + 1e-5) + be_ref[...]
    y = jnp.maximum(y, 0.0)
    h = jnp.dot(y, wb_ref[...], preferred_element_type=jnp.float32) + bb_ref[...]
    h = jnp.maximum(h, 0.0)
    u = jnp.dot(h, w3a_ref[...], preferred_element_type=jnp.float32) + b3a_ref[...]
    u = jnp.maximum(u, 0.0)
    logits = (jnp.dot(u, w3b_ref[...], preferred_element_type=jnp.float32)
              + b3b_ref[...])
    m = jnp.max(logits, axis=-1, keepdims=True)
    lse = jnp.log(jnp.sum(jnp.exp(logits - m), axis=-1, keepdims=True)) + m
    out_ref[...] = logits - lse


_dense2 = pl.pallas_call(
    _dense2_body,
    out_shape=jax.ShapeDtypeStruct((N, D_OUT), jnp.float32),
)


def kernel(x, edge_index, W1a, b1a, g1, be1, W1b, b1b,
           W2a, b2a, g2, be2, W2b, b2b, W3a, b3a, W3b, b3b):
    pad = EPW_P - EPW
    pad_src = jnp.broadcast_to(jnp.arange(pad, dtype=jnp.int32), (NW, pad))
    src = jnp.concatenate([edge_index[0].reshape(NW, EPW), pad_src], axis=1)
    pad_dst = jnp.broadcast_to(N + jnp.arange(pad, dtype=jnp.int32),
                               (NW, pad))
    dst = jnp.concatenate([edge_index[1].reshape(NW, EPW), pad_dst], axis=1)
    src = src.reshape(NW, NWIN, W, C)
    dst = dst.reshape(NW, NWIN, W, 2, C // 2)
    zeros = jnp.zeros((ROWS_PT, D_H), jnp.float32)

    _agg = _make_agg()
    p1 = _agg(src, dst, x, zeros)
    h1 = _dense1(x, p1, W1a, b1a.reshape(1, -1), g1.reshape(1, -1),
                 be1.reshape(1, -1), W1b, b1b.reshape(1, -1))
    p2 = _agg(src, dst, h1, zeros)
    out = _dense2(h1, p2, W2a, b2a.reshape(1, -1), g2.reshape(1, -1),
                  be2.reshape(1, -1), W2b, b2b.reshape(1, -1),
                  W3a, b3a.reshape(1, -1), W3b, b3b.reshape(1, -1))
    return out


# speedup vs baseline: 1.0036x; 1.0036x over previous
import functools

import jax
import jax.numpy as jnp
from jax import lax
from jax.experimental import pallas as pl
from jax.experimental.pallas import tpu as pltpu
from jax.experimental.pallas import tpu_sc as plsc

N = 10000
E = 320000
D_IN = 128
D_H = 128
D_OUT = 64

NC = 2
NS = 16
NW = NC * NS
EPW = E // NW
C = 128
EPW_P = 10240
CPW = EPW_P // C
W = 20
NWIN = CPW // W
N_PAD = 10240
ROWS_PT = N_PAD // NS

@functools.cache
def _make_agg():
    mesh = plsc.VectorSubcoreMesh(core_axis_name="c", subcore_axis_name="s")

    @functools.partial(
        pl.kernel,
        out_type=jax.ShapeDtypeStruct((NC * N_PAD, D_H), jnp.float32),
        mesh=mesh,
        scratch_types=[
            pltpu.VMEM((2, W, C), jnp.int32),
            pltpu.VMEM((2, W, C), jnp.int32),
            pltpu.VMEM((2, C, D_H), jnp.float32),
            pltpu.VMEM_SHARED((N_PAD, D_H), jnp.float32),
            pltpu.SemaphoreType.DMA,
            pltpu.SemaphoreType.DMA,
            pltpu.SemaphoreType.DMA,
            pltpu.SemaphoreType.DMA,
        ],
    )
    def _agg(src_hbm, dst_hbm, h_hbm, zeros_hbm, out_hbm,
             srcv, dstv, rows, acc, sem0, sem1, semi, semz):
        cid = lax.axis_index("c")
        sid = lax.axis_index("s")
        wid = cid * NS + sid

        pltpu.async_copy(zeros_hbm, acc.at[pl.ds(sid * ROWS_PT, ROWS_PT)],
                         semz)
        pltpu.async_copy(src_hbm.at[wid, 0], srcv.at[0], semi)
        pltpu.async_copy(dst_hbm.at[wid, 0], dstv.at[0], semi)
        pltpu.make_async_copy(src_hbm.at[wid, 0], srcv.at[0], semi).wait()
        pltpu.make_async_copy(dst_hbm.at[wid, 0], dstv.at[0], semi).wait()
        pltpu.async_copy(h_hbm.at[srcv.at[0].at[0]], rows.at[0], sem0)
        pltpu.async_copy(h_hbm.at[srcv.at[0].at[1]], rows.at[1], sem1)
        pltpu.make_async_copy(zeros_hbm,
                              acc.at[pl.ds(sid * ROWS_PT, ROWS_PT)],
                              semz).wait()
        plsc.subcore_barrier()

        for w in range(NWIN):
            pb = w % 2
            sw = srcv.at[pb]
            dw = dstv.at[pb]
            if w >= 1:
                pltpu.make_async_copy(src_hbm.at[wid, w], sw, semi).wait()
                pltpu.make_async_copy(dst_hbm.at[wid, w], dw, semi).wait()
            if w + 1 < NWIN:
                nb = (w + 1) % 2
                pltpu.async_copy(src_hbm.at[wid, w + 1], srcv.at[nb], semi)
                pltpu.async_copy(dst_hbm.at[wid, w + 1], dstv.at[nb], semi)

            if w >= 1:
                pltpu.async_copy(h_hbm.at[sw.at[0]], rows.at[0], sem0)
                pltpu.async_copy(h_hbm.at[sw.at[1]], rows.at[1], sem1)

            def body(i, carry, sw=sw, dw=dw):
                j0 = 2 * i
                j1 = j0 + 1
                pltpu.make_async_copy(h_hbm.at[sw.at[j0]], rows.at[0],
                                      sem0).wait()
                pltpu.sync_copy(rows.at[0], acc.at[dw.at[j0]], add=True)
                pltpu.async_copy(h_hbm.at[sw.at[j0 + 2]], rows.at[0], sem0)
                pltpu.make_async_copy(h_hbm.at[sw.at[j1]], rows.at[1],
                                      sem1).wait()
                pltpu.sync_copy(rows.at[1], acc.at[dw.at[j1]], add=True)
                pltpu.async_copy(h_hbm.at[sw.at[j1 + 2]], rows.at[1], sem1)
                return carry

            n_steady = (W - 2) // 2
            lax.fori_loop(0, n_steady, body, 0)
            j_a = 2 * n_steady
            pltpu.make_async_copy(h_hbm.at[sw.at[j_a]], rows.at[0],
                                  sem0).wait()
            pltpu.sync_copy(rows.at[0], acc.at[dw.at[j_a]], add=True)
            if W % 2:
                pltpu.async_copy(h_hbm.at[sw.at[W - 1]], rows.at[0], sem0)
            pltpu.make_async_copy(h_hbm.at[sw.at[j_a + 1]], rows.at[1],
                                  sem1).wait()
            pltpu.sync_copy(rows.at[1], acc.at[dw.at[j_a + 1]], add=True)
            if W % 2:
                pltpu.make_async_copy(h_hbm.at[sw.at[W - 1]], rows.at[0],
                                      sem0).wait()
                pltpu.sync_copy(rows.at[0], acc.at[dw.at[W - 1]], add=True)
        plsc.subcore_barrier()
        pltpu.sync_copy(acc.at[pl.ds(sid * ROWS_PT, ROWS_PT)],
                        out_hbm.at[pl.ds(cid * N_PAD + sid * ROWS_PT, ROWS_PT)])

    return _agg


def _dense1_body(x_ref, p_ref, wa_ref, ba_ref, g_ref, be_ref, wb_ref, bb_ref,
                 out_ref):
    z = x_ref[...] + p_ref[0:N, :] + p_ref[N_PAD:N_PAD + N, :]
    t = jnp.dot(z, wa_ref[...], preferred_element_type=jnp.float32) + ba_ref[...]
    mu = jnp.mean(t, axis=0, keepdims=True)
    var = jnp.mean((t - mu) ** 2, axis=0, keepdims=True)
    y = g_ref[...] * (t - mu) * lax.rsqrt(var + 1e-5) + be_ref[...]
    y = jnp.maximum(y, 0.0)
    h = jnp.dot(y, wb_ref[...], preferred_element_type=jnp.float32) + bb_ref[...]
    out_ref[...] = jnp.maximum(h, 0.0)


_dense1 = pl.pallas_call(
    _dense1_body,
    out_shape=jax.ShapeDtypeStruct((N, D_H), jnp.float32),
)


def _dense2_body(x_ref, p_ref, wa_ref, ba_ref, g_ref, be_ref, wb_ref, bb_ref,
                 w3a_ref, b3a_ref, w3b_ref, b3b_ref, out_ref):
    z = x_ref[...] + p_ref[0:N, :] + p_ref[N_PAD:N_PAD + N, :]
    t = jnp.dot(z, wa_ref[...], preferred_element_type=jnp.float32) + ba_ref[...]
    mu = jnp.mean(t, axis=0, keepdims=True)
    var = jnp.mean((t - mu) ** 2, axis=0, keepdims=True)
    y = g_ref[...] * (t - mu) * lax.rsqrt(var + 1e-5) + be_ref[...]
    y = jnp.maximum(y, 0.0)
    h = jnp.dot(y, wb_ref[...], preferred_element_type=jnp.float32) + bb_ref[...]
    h = jnp.maximum(h, 0.0)
    u = jnp.dot(h, w3a_ref[...], preferred_element_type=jnp.float32) + b3a_ref[...]
    u = jnp.maximum(u, 0.0)
    logits = (jnp.dot(u, w3b_ref[...], preferred_element_type=jnp.float32)
              + b3b_ref[...])
    m = jnp.max(logits, axis=-1, keepdims=True)
    lse = jnp.log(jnp.sum(jnp.exp(logits - m), axis=-1, keepdims=True)) + m
    out_ref[...] = logits - lse


_dense2 = pl.pallas_call(
    _dense2_body,
    out_shape=jax.ShapeDtypeStruct((N, D_OUT), jnp.float32),
)


def kernel(x, edge_index, W1a, b1a, g1, be1, W1b, b1b,
           W2a, b2a, g2, be2, W2b, b2b, W3a, b3a, W3b, b3b):
    pad = EPW_P - EPW
    pad_src = jnp.broadcast_to(jnp.arange(pad, dtype=jnp.int32), (NW, pad))
    src = jnp.concatenate([edge_index[0].reshape(NW, EPW), pad_src], axis=1)
    pad_dst = jnp.broadcast_to(N + jnp.arange(pad, dtype=jnp.int32),
                               (NW, pad))
    dst = jnp.concatenate([edge_index[1].reshape(NW, EPW), pad_dst], axis=1)
    src = src.reshape(NW, NWIN, W, C)
    dst = dst.reshape(NW, NWIN, W, C)
    zeros = jnp.zeros((ROWS_PT, D_H), jnp.float32)

    _agg = _make_agg()
    p1 = _agg(src, dst, x, zeros)
    h1 = _dense1(x, p1, W1a, b1a.reshape(1, -1), g1.reshape(1, -1),
                 be1.reshape(1, -1), W1b, b1b.reshape(1, -1))
    p2 = _agg(src, dst, h1, zeros)
    out = _dense2(h1, p2, W2a, b2a.reshape(1, -1), g2.reshape(1, -1),
                  be2.reshape(1, -1), W2b, b2b.reshape(1, -1),
                  W3a, b3a.reshape(1, -1), W3b, b3b.reshape(1, -1))
    return out
